# baseline (speedup 1.0000x reference)
import functools

import jax
import jax.numpy as jnp
import numpy as np
from jax import lax
from jax.experimental import pallas as pl
from jax.experimental.pallas import tpu as pltpu
from jax.experimental.pallas import tpu_sc as plsc

_LOG2 = 0.6931471805599453

_NC = 2
_NS = 16
_LANES = 16

_CHUNK = 128
_CHUNKS_PER_TILE = 80
_GRP = 16

_ACC_ROWS = 10240

_EPAD = _NC * _NS * _CHUNKS_PER_TILE * _CHUNK
_NPAD = 10240
_EBLK = 2048
_HIMASK = -65536


def _rf_body(x_ref, w_ref, o_ref):
    o_ref[:, :] = jnp.dot(x_ref[:, :], w_ref[:, :],
                          preferred_element_type=jnp.float32)


def _filter_body(e_ref, off_ref, wid_ref, w2_ref, b2_ref, o_ref,
                 *, ecount):
    d = (e_ref[:, :] - off_ref[:, :]) / wid_ref[:, :]
    es = jnp.exp(-0.5 * d * d)
    w = jnp.dot(es, w2_ref[:, :],
                preferred_element_type=jnp.float32) + b2_ref[:, :]
    gi = pl.program_id(0) * _EBLK + lax.broadcasted_iota(jnp.int32, (_EBLK, 1), 0)
    w = jnp.where(gi < ecount, w, 0.0)
    lo = w[:, :64].astype(jnp.bfloat16).astype(jnp.float32)
    hi = w[:, 64:].astype(jnp.bfloat16).astype(jnp.float32)
    lo_b = lax.shift_right_logical(lax.bitcast_convert_type(lo, jnp.int32), 16)
    hi_b = lax.bitcast_convert_type(hi, jnp.int32) & _HIMASK
    o_ref[:, :] = hi_b | lo_b


def _out_mlp_body(p0_ref, p1_ref, d1_ref, b1_ref, d2_ref, b2_ref, o_ref):
    h = p0_ref[:, :] + p1_ref[:, :]
    t = jnp.dot(h, d1_ref[:, :], preferred_element_type=jnp.float32) + b1_ref[:, :]
    m = jnp.maximum(t, 0.0)
    sp = m + jnp.log(jnp.exp(t - m) + jnp.exp(-m)) - _LOG2
    o_ref[:, :] = jnp.dot(sp, d2_ref[:, :],
                          preferred_element_type=jnp.float32) + b2_ref[:, :]


def _sc_body(rf_hbm, wp_hbm, dst_hbm, sidx_hbm, out_hbm,
             sidx_v, dst_v, g0, g1, w_v, acc, semg, semw, sems):
    c = lax.axis_index("c")
    s = lax.axis_index("s")
    wid = c * _NS + s
    R = _CHUNKS_PER_TILE
    tile_row = wid * R
    dr = _ACC_ROWS // _NS
    full = dr // _CHUNK

    def _zero_row(i, _):
        for k in range(8):
            g0[i, pl.ds(k * _LANES, _LANES)] = jnp.zeros((_LANES,), jnp.float32)
        return 0
    lax.fori_loop(0, _CHUNK, _zero_row, 0)
    for t in range(full):
        pltpu.sync_copy(g0, acc.at[pl.ds(s * dr + t * _CHUNK, _CHUNK)])
    plsc.subcore_barrier()

    def _gather_row(j, buf):
        return pltpu.async_copy(rf_hbm.at[sidx_v.at[j]], buf, semg)

    def _wload(i):
        return pltpu.async_copy(
            wp_hbm.at[pl.ds((tile_row + i) * (_CHUNK // 2), _CHUNK // 2)],
            w_v, semw)

    def _mul(gb):
        himask = jnp.full((_LANES,), _HIMASK, jnp.int32)

        def _mul_row(t, _):
            for half in range(2):
                r = 2 * t + half
                for k in range(4):
                    u = w_v[t, pl.ds(64 * half + k * _LANES, _LANES)]
                    wlo = lax.bitcast_convert_type(
                        lax.shift_left(u, 16), jnp.float32)
                    whi = lax.bitcast_convert_type(u & himask, jnp.float32)
                    slo = pl.ds(32 * k, _LANES)
                    shi = pl.ds(32 * k + _LANES, _LANES)
                    gb[r, slo] = gb[r, slo] * wlo
                    gb[r, shi] = gb[r, shi] * whi
            return 0
        lax.fori_loop(0, _CHUNK // 2, _mul_row, 0)

    def _wait_gather(gb):
        pltpu.make_async_copy(rf_hbm.at[sidx_v.at[0]], gb, semg).wait()

    def _wait_w():
        pltpu.make_async_copy(wp_hbm.at[pl.ds(0, _CHUNK // 2)], w_v, semw).wait()

    def _drain_scatter(go):
        pltpu.make_async_copy(go, acc.at[dst_v.at[0]], sems).wait()

    def _scatter(j, gb):
        pltpu.async_copy(gb, acc.at[dst_v.at[j]], sems, add=True)

    def _phase(i, j, gb, go, last):
        _wait_gather(gb)
        _drain_scatter(go)
        if not last:
            _gather_row(j + 1, go)
        _wait_w()
        _mul(gb)
        if not last:
            _wload(i + 1)
        _scatter(j, gb)

    for g in range(_CHUNKS_PER_TILE // _GRP):
        i0 = g * _GRP
        if g > 0:
            _drain_scatter(g1)
        base = pl.multiple_of(tile_row + i0, 8)
        pltpu.sync_copy(sidx_hbm.at[pl.ds(base, _GRP)], sidx_v)
        pltpu.sync_copy(dst_hbm.at[pl.ds(base, _GRP)], dst_v)
        _gather_row(0, g0)
        if g == 0:
            _wload(0)

        _wait_gather(g0)
        _gather_row(1, g1)
        _wait_w()
        _mul(g0)
        _wload(i0 + 1)
        _scatter(0, g0)

        def _pair(t, _):
            j = 1 + 2 * t
            i = i0 + j
            _phase(i, j, g1, g0, False)
            _phase(i + 1, j + 1, g0, g1, False)
            return 0

        lax.fori_loop(0, (_GRP - 2) // 2, _pair, 0)

        _wait_gather(g1)
        _drain_scatter(g0)
        _wait_w()
        _mul(g1)
        if g + 1 < _CHUNKS_PER_TILE // _GRP:
            _wload(i0 + _GRP)
        _scatter(_GRP - 1, g1)
    _drain_scatter(g1)

    plsc.subcore_barrier()
    out_base = c * _ACC_ROWS + s * dr
    for t in range(full):
        pltpu.sync_copy(acc.at[pl.ds(s * dr + t * _CHUNK, _CHUNK)], g0)
        pltpu.sync_copy(g0, out_hbm.at[pl.ds(out_base + t * _CHUNK, _CHUNK)])


def kernel(r, e, a, offsets, widths, df1_W, df1_b, df2_W, df2_b, atom_W,
           d1_W, d1_b, d2_W, d2_b):
    n, nab = r.shape
    nf = atom_W.shape[1]
    ng = offsets.shape[0]
    e_count = e.shape[0]

    r_p = jnp.concatenate([r, jnp.zeros((_NPAD - n, nab), jnp.float32)])
    rf = pl.pallas_call(
        _rf_body,
        grid=(_NPAD // _EBLK,),
        in_specs=[
            pl.BlockSpec((_EBLK, nab), lambda i: (i, 0)),
            pl.BlockSpec((nab, nf), lambda i: (0, 0)),
        ],
        out_specs=pl.BlockSpec((_EBLK, nf), lambda i: (i, 0)),
        out_shape=jax.ShapeDtypeStruct((_NPAD, nf), jnp.float32),
    )(r_p, atom_W)

    cperm = np.concatenate([np.arange(16) + 32 * k for k in range(4)] +
                           [np.arange(16) + 32 * k + 16 for k in range(4)])
    cperm = jnp.asarray(cperm.astype(np.int32))

    gpad = 128
    off_p = jnp.concatenate([offsets, jnp.zeros((gpad - ng,), jnp.float32)])[None, :]
    wid_p = jnp.concatenate([widths, jnp.ones((gpad - ng,), jnp.float32)])[None, :]
    w2_p = jnp.concatenate(
        [df2_W, jnp.zeros((gpad - ng, nf), jnp.float32)], axis=0)[:, cperm]
    b2_p = df2_b[cperm]
    e_p = jnp.concatenate(
        [e[:, 0], jnp.zeros((_EPAD - e_count,), jnp.float32)])[:, None]

    wp = pl.pallas_call(
        functools.partial(_filter_body, ecount=e_count),
        grid=(_EPAD // _EBLK,),
        in_specs=[
            pl.BlockSpec((_EBLK, 1), lambda i: (i, 0)),
            pl.BlockSpec((1, gpad), lambda i: (0, 0)),
            pl.BlockSpec((1, gpad), lambda i: (0, 0)),
            pl.BlockSpec((gpad, nf), lambda i: (0, 0)),
            pl.BlockSpec((1, nf), lambda i: (0, 0)),
        ],
        out_specs=pl.BlockSpec((_EBLK, nf // 2), lambda i: (i, 0)),
        out_shape=jax.ShapeDtypeStruct((_EPAD, nf // 2), jnp.int32),
    )(e_p, off_p, wid_p, w2_p, b2_p[None, :])

    pad = _EPAD - e_count
    dst = jnp.concatenate(
        [a[:, 0], jnp.zeros((pad,), jnp.int32)]).reshape(-1, _CHUNK)
    sidx = jnp.concatenate(
        [a[:, 1], jnp.zeros((pad,), jnp.int32)]).reshape(-1, _CHUNK)

    sc_fn = pl.kernel(
        _sc_body,
        out_type=jax.ShapeDtypeStruct((_NC * _ACC_ROWS, nf), jnp.float32),
        mesh=plsc.VectorSubcoreMesh(core_axis_name="c", subcore_axis_name="s",
                                    num_cores=_NC),
        scratch_types=[
            pltpu.VMEM((_GRP, _CHUNK), jnp.int32),
            pltpu.VMEM((_GRP, _CHUNK), jnp.int32),
            pltpu.VMEM((_CHUNK, nf), jnp.float32),
            pltpu.VMEM((_CHUNK, nf), jnp.float32),
            pltpu.VMEM((_CHUNK // 2, nf), jnp.int32),
            pltpu.VMEM_SHARED((_ACC_ROWS, nf), jnp.float32),
            pltpu.SemaphoreType.DMA,
            pltpu.SemaphoreType.DMA,
            pltpu.SemaphoreType.DMA,
        ],
    )
    partials = sc_fn(rf, wp.reshape(-1, nf), dst, sidx)

    p0 = partials[0:n]
    p1 = partials[_ACC_ROWS:_ACC_ROWS + n]
    rblk = 1000
    out = pl.pallas_call(
        _out_mlp_body,
        grid=(n // rblk,),
        in_specs=[
            pl.BlockSpec((rblk, nf), lambda i: (i, 0)),
            pl.BlockSpec((rblk, nf), lambda i: (i, 0)),
            pl.BlockSpec((nf, nab), lambda i: (0, 0)),
            pl.BlockSpec((1, nab), lambda i: (0, 0)),
            pl.BlockSpec((nab, nab), lambda i: (0, 0)),
            pl.BlockSpec((1, nab), lambda i: (0, 0)),
        ],
        out_specs=pl.BlockSpec((rblk, nab), lambda i: (i, 0)),
        out_shape=jax.ShapeDtypeStruct((n, nab), jnp.float32),
    )(p0, p1, d1_W, d1_b[None, :], d2_W, d2_b[None, :])
    return out
